# Initial kernel scaffold; baseline (speedup 1.0000x reference)
import functools

import jax
import jax.numpy as jnp
from jax import lax
from jax.experimental import pallas as pl
from jax.experimental.pallas import tpu as pltpu
from jax.experimental.pallas import tpu_sc as plsc

_ALPHA = 0.1
_HOP = 2
_NLAYERS = 4

_NC = 2
_NS = 16
_NTILES = _NC * _NS
_CH = 128


def _sc_mesh():
    return plsc.VectorSubcoreMesh(
        core_axis_name="c", subcore_axis_name="s",
        num_cores=_NC, num_subcores=_NS)


@functools.lru_cache(maxsize=None)
def _build_deg(npad, epad):
    ept = epad // _NTILES
    iters = ept // _CH
    rows_pt = npad // _NS

    def body(dst_hbm, out_hbm, didx_v, ones_v, zbuf_v, acc_sh):
        c = lax.axis_index("c")
        s = lax.axis_index("s")
        w = c * _NS + s
        one16 = jnp.full((16,), 1.0, jnp.float32)
        zero16 = jnp.zeros((16,), jnp.float32)
        for j in range(_CH // 16):
            ones_v[pl.ds(j * 16, 16)] = one16

        def zb(i, carry):
            zbuf_v[pl.ds(i * 16, 16)] = zero16
            return carry

        lax.fori_loop(0, rows_pt // 16, zb, 0)
        r0 = s * rows_pt
        pltpu.sync_copy(zbuf_v, acc_sh.at[pl.ds(r0, rows_pt)])
        plsc.subcore_barrier()

        def step(it, carry):
            base = pl.multiple_of(w * ept + it * _CH, _CH)
            pltpu.sync_copy(dst_hbm.at[pl.ds(base, _CH)], didx_v)
            pltpu.sync_copy(ones_v, acc_sh.at[didx_v], add=True)
            return carry

        lax.fori_loop(0, iters, step, 0)
        plsc.subcore_barrier()
        pltpu.sync_copy(acc_sh.at[pl.ds(r0, rows_pt)],
                        out_hbm.at[c, pl.ds(r0, rows_pt)])

    return pl.kernel(
        body,
        out_type=jax.ShapeDtypeStruct((_NC, npad), jnp.float32),
        mesh=_sc_mesh(),
        scratch_types=[
            pltpu.VMEM((_CH,), jnp.int32),
            pltpu.VMEM((_CH,), jnp.float32),
            pltpu.VMEM((rows_pt,), jnp.float32),
            pltpu.VMEM_SHARED((npad,), jnp.float32),
        ],
    )


@functools.lru_cache(maxsize=None)
def _build_spmm(npad, epad, nhid):
    ept = epad // _NTILES
    iters = ept // _CH
    rows_pt = npad // _NS

    def body(g_hbm, src_hbm, dst_hbm, out_hbm, sidx_v, didx_v, rows_v,
             acc_sh, sem):
        c = lax.axis_index("c")
        s = lax.axis_index("s")
        w = c * _NS + s
        zero16 = jnp.zeros((16,), jnp.float32)

        def zb(i, carry):
            for j in range(nhid // 16):
                rows_v[i, pl.ds(j * 16, 16)] = zero16
            return carry

        lax.fori_loop(0, _CH, zb, 0)
        r0 = s * rows_pt
        for k in range(rows_pt // _CH):
            pltpu.sync_copy(rows_v, acc_sh.at[pl.ds(r0 + k * _CH, _CH)])
        plsc.subcore_barrier()

        def step(it, carry):
            base = pl.multiple_of(w * ept + it * _CH, _CH)
            pltpu.sync_copy(src_hbm.at[pl.ds(base, _CH)], sidx_v)
            pltpu.sync_copy(dst_hbm.at[pl.ds(base, _CH)], didx_v)
            pltpu.async_copy(g_hbm.at[sidx_v], rows_v, sem).wait()
            pltpu.sync_copy(rows_v, acc_sh.at[didx_v], add=True)
            return carry

        lax.fori_loop(0, iters, step, 0)
        plsc.subcore_barrier()
        pltpu.sync_copy(acc_sh.at[pl.ds(r0, rows_pt)],
                        out_hbm.at[c, pl.ds(r0, rows_pt)])

    return pl.kernel(
        body,
        out_type=jax.ShapeDtypeStruct((_NC, npad, nhid), jnp.float32),
        mesh=_sc_mesh(),
        scratch_types=[
            pltpu.VMEM((_CH,), jnp.int32),
            pltpu.VMEM((_CH,), jnp.int32),
            pltpu.VMEM((_CH, nhid), jnp.float32),
            pltpu.VMEM_SHARED((npad, nhid), jnp.float32),
            pltpu.SemaphoreType.DMA,
        ],
    )


def _dinv_from(degT):
    deg = jnp.sum(degT, axis=1, keepdims=True) + 1.0
    return lax.rsqrt(jnp.maximum(deg, 1.0))


@functools.lru_cache(maxsize=None)
def _build_fc0(npad, nfeat, nhid):
    def body(x_ref, w_ref, b_ref, degT_ref, h_ref, g_ref):
        h = jnp.maximum(
            jnp.dot(x_ref[...], w_ref[...],
                    preferred_element_type=jnp.float32) + b_ref[...], 0.0)
        dinv = _dinv_from(degT_ref[...])
        h_ref[...] = h
        g_ref[...] = h * dinv

    return pl.pallas_call(
        body,
        out_shape=[jax.ShapeDtypeStruct((npad, nhid), jnp.float32)] * 2)


@functools.lru_cache(maxsize=None)
def _build_mid(npad, nhid):
    def body(p_ref, g_ref, degT_ref, o_ref):
        deg = jnp.sum(degT_ref[...], axis=1, keepdims=True) + 1.0
        inv = 1.0 / jnp.maximum(deg, 1.0)
        o_ref[...] = (p_ref[0] + p_ref[1] + g_ref[...]) * inv

    return pl.pallas_call(
        body,
        out_shape=jax.ShapeDtypeStruct((npad, nhid), jnp.float32))


@functools.lru_cache(maxsize=None)
def _build_layer(npad, nhid):
    def body(p_ref, g_ref, degT_ref, h0_ref, wc_ref, h_ref, gn_ref):
        dinv = _dinv_from(degT_ref[...])
        hi = (p_ref[0] + p_ref[1] + g_ref[...]) * dinv
        sup = (1.0 - _ALPHA) * hi + _ALPHA * h0_ref[...]
        h = jnp.maximum(
            jnp.dot(sup, wc_ref[...], preferred_element_type=jnp.float32),
            0.0)
        h_ref[...] = h
        gn_ref[...] = h * dinv

    return pl.pallas_call(
        body,
        out_shape=[jax.ShapeDtypeStruct((npad, nhid), jnp.float32)] * 2)


@functools.lru_cache(maxsize=None)
def _build_final(npad, nhid, ncpad):
    def body(h_ref, wf_ref, bf_ref, o_ref):
        logits = jnp.dot(h_ref[...], wf_ref[...],
                         preferred_element_type=jnp.float32) + bf_ref[...]
        m = jnp.max(logits, axis=1, keepdims=True)
        sh = logits - m
        lse = jnp.log(jnp.sum(jnp.exp(sh), axis=1, keepdims=True))
        o_ref[...] = sh - lse

    return pl.pallas_call(
        body,
        out_shape=jax.ShapeDtypeStruct((npad, ncpad), jnp.float32))


def kernel(x, edge_index, W_fc0, b_fc0, W_conv, W_fc1, b_fc1):
    n, nfeat = x.shape
    nhid = W_fc0.shape[1]
    ncls = W_fc1.shape[1]
    e = edge_index.shape[1]
    npad = -(-n // (_NS * _CH)) * (_NS * _CH)
    epad = -(-e // (_NTILES * _CH)) * (_NTILES * _CH)

    src = edge_index[0]
    dst = edge_index[1]
    if epad != e:
        fill = jnp.full((epad - e,), npad - 1, dtype=jnp.int32)
        src = jnp.concatenate([src, fill])
        dst = jnp.concatenate([dst, fill])
    xp = jnp.zeros((npad, nfeat), x.dtype).at[:n].set(x)

    deg = _build_deg(npad, epad)(dst)
    degT = deg.T

    h, g = _build_fc0(npad, nfeat, nhid)(
        xp, W_fc0, b_fc0.reshape(1, nhid), degT)
    h0 = h
    spmm = _build_spmm(npad, epad, nhid)
    mid = _build_mid(npad, nhid)
    layer = _build_layer(npad, nhid)
    for _ in range(_NLAYERS):
        gi = g
        for _ in range(_HOP - 1):
            p = spmm(gi, src, dst)
            gi = mid(p, gi, degT)
        p = spmm(gi, src, dst)
        h, g = layer(p, gi, degT, h0, W_conv)

    ncpad = -(-ncls // 64) * 64
    Wf = jnp.zeros((nhid, ncpad), W_fc1.dtype).at[:, :ncls].set(W_fc1)
    bf = jnp.full((1, ncpad), -1e30, jnp.float32).at[0, :ncls].set(b_fc1)
    out = _build_final(npad, nhid, ncpad)(h, Wf, bf)
    return out[:n, :ncls]


# speedup vs baseline: 9.9046x; 9.9046x over previous
import functools

import jax
import jax.numpy as jnp
from jax import lax
from jax.experimental import pallas as pl
from jax.experimental.pallas import tpu as pltpu
from jax.experimental.pallas import tpu_sc as plsc

_ALPHA = 0.1
_HOP = 2
_NLAYERS = 4

_NC = 2
_NS = 16
_NTILES = _NC * _NS
_CH = 128


def _sc_mesh():
    return plsc.VectorSubcoreMesh(
        core_axis_name="c", subcore_axis_name="s",
        num_cores=_NC, num_subcores=_NS)


@functools.lru_cache(maxsize=None)
def _build_deg(npad, epad):
    ept = epad // _NTILES
    iters = ept // _CH
    rows_pt = npad // _NS

    def body(dst_hbm, out_hbm, didx_v, ones_v, zbuf_v, acc_sh):
        c = lax.axis_index("c")
        s = lax.axis_index("s")
        w = c * _NS + s
        one16 = jnp.full((16,), 1.0, jnp.float32)
        zero16 = jnp.zeros((16,), jnp.float32)
        for j in range(_CH // 16):
            ones_v[pl.ds(j * 16, 16)] = one16

        def zb(i, carry):
            zbuf_v[pl.ds(i * 16, 16)] = zero16
            return carry

        lax.fori_loop(0, rows_pt // 16, zb, 0)
        r0 = s * rows_pt
        pltpu.sync_copy(zbuf_v, acc_sh.at[pl.ds(r0, rows_pt)])
        plsc.subcore_barrier()

        def step(it, carry):
            base = pl.multiple_of(w * ept + it * _CH, _CH)
            pltpu.sync_copy(dst_hbm.at[pl.ds(base, _CH)], didx_v)
            pltpu.sync_copy(ones_v, acc_sh.at[didx_v], add=True)
            return carry

        lax.fori_loop(0, iters, step, 0)
        plsc.subcore_barrier()
        pltpu.sync_copy(acc_sh.at[pl.ds(r0, rows_pt)],
                        out_hbm.at[c, pl.ds(r0, rows_pt)])

    return pl.kernel(
        body,
        out_type=jax.ShapeDtypeStruct((_NC, npad), jnp.float32),
        mesh=_sc_mesh(),
        compiler_params=pltpu.CompilerParams(use_tc_tiling_on_sc=False),
        scratch_types=[
            pltpu.VMEM((_CH,), jnp.int32),
            pltpu.VMEM((_CH,), jnp.float32),
            pltpu.VMEM((rows_pt,), jnp.float32),
            pltpu.VMEM_SHARED((npad,), jnp.float32),
        ],
    )


@functools.lru_cache(maxsize=None)
def _build_spmm(npad, epad, nhid):
    ept = epad // _NTILES
    iters = ept // _CH
    rows_pt = npad // _NS

    def body(g_hbm, src_hbm, dst_hbm, out_hbm, sidx_v, didx_v, rows_v,
             acc_sh, sem):
        c = lax.axis_index("c")
        s = lax.axis_index("s")
        w = c * _NS + s
        zero16 = jnp.zeros((16,), jnp.float32)

        def zb(i, carry):
            for j in range(nhid // 16):
                rows_v[i, pl.ds(j * 16, 16)] = zero16
            return carry

        lax.fori_loop(0, _CH, zb, 0)
        r0 = s * rows_pt
        for k in range(rows_pt // _CH):
            pltpu.sync_copy(rows_v, acc_sh.at[pl.ds(r0 + k * _CH, _CH)])
        plsc.subcore_barrier()

        def step(it, carry):
            base = pl.multiple_of(w * ept + it * _CH, _CH)
            pltpu.sync_copy(src_hbm.at[pl.ds(base, _CH)], sidx_v)
            pltpu.sync_copy(dst_hbm.at[pl.ds(base, _CH)], didx_v)
            pltpu.async_copy(g_hbm.at[sidx_v], rows_v, sem).wait()
            pltpu.sync_copy(rows_v, acc_sh.at[didx_v], add=True)
            return carry

        lax.fori_loop(0, iters, step, 0)
        plsc.subcore_barrier()
        pltpu.sync_copy(acc_sh.at[pl.ds(r0, rows_pt)],
                        out_hbm.at[c, pl.ds(r0, rows_pt)])

    return pl.kernel(
        body,
        out_type=jax.ShapeDtypeStruct((_NC, npad, nhid), jnp.float32),
        mesh=_sc_mesh(),
        compiler_params=pltpu.CompilerParams(use_tc_tiling_on_sc=False),
        scratch_types=[
            pltpu.VMEM((_CH,), jnp.int32),
            pltpu.VMEM((_CH,), jnp.int32),
            pltpu.VMEM((_CH, nhid), jnp.float32),
            pltpu.VMEM_SHARED((npad, nhid), jnp.float32),
            pltpu.SemaphoreType.DMA,
        ],
    )


def _dinv_from(degT):
    deg = jnp.sum(degT, axis=1, keepdims=True) + 1.0
    return lax.rsqrt(jnp.maximum(deg, 1.0))


@functools.lru_cache(maxsize=None)
def _build_fc0(npad, nfeat, nhid):
    def body(x_ref, w_ref, b_ref, degT_ref, h_ref, g_ref):
        h = jnp.maximum(
            jnp.dot(x_ref[...], w_ref[...],
                    preferred_element_type=jnp.float32) + b_ref[...], 0.0)
        dinv = _dinv_from(degT_ref[...])
        h_ref[...] = h
        g_ref[...] = h * dinv

    return pl.pallas_call(
        body,
        out_shape=[jax.ShapeDtypeStruct((npad, nhid), jnp.float32)] * 2)


@functools.lru_cache(maxsize=None)
def _build_mid(npad, nhid):
    def body(p_ref, g_ref, degT_ref, o_ref):
        deg = jnp.sum(degT_ref[...], axis=1, keepdims=True) + 1.0
        inv = 1.0 / jnp.maximum(deg, 1.0)
        o_ref[...] = (p_ref[0] + p_ref[1] + g_ref[...]) * inv

    return pl.pallas_call(
        body,
        out_shape=jax.ShapeDtypeStruct((npad, nhid), jnp.float32))


@functools.lru_cache(maxsize=None)
def _build_layer(npad, nhid):
    def body(p_ref, g_ref, degT_ref, h0_ref, wc_ref, h_ref, gn_ref):
        dinv = _dinv_from(degT_ref[...])
        hi = (p_ref[0] + p_ref[1] + g_ref[...]) * dinv
        sup = (1.0 - _ALPHA) * hi + _ALPHA * h0_ref[...]
        h = jnp.maximum(
            jnp.dot(sup, wc_ref[...], preferred_element_type=jnp.float32),
            0.0)
        h_ref[...] = h
        gn_ref[...] = h * dinv

    return pl.pallas_call(
        body,
        out_shape=[jax.ShapeDtypeStruct((npad, nhid), jnp.float32)] * 2)


@functools.lru_cache(maxsize=None)
def _build_final(npad, nhid, ncpad):
    def body(h_ref, wf_ref, bf_ref, o_ref):
        logits = jnp.dot(h_ref[...], wf_ref[...],
                         preferred_element_type=jnp.float32) + bf_ref[...]
        m = jnp.max(logits, axis=1, keepdims=True)
        sh = logits - m
        lse = jnp.log(jnp.sum(jnp.exp(sh), axis=1, keepdims=True))
        o_ref[...] = sh - lse

    return pl.pallas_call(
        body,
        out_shape=jax.ShapeDtypeStruct((npad, ncpad), jnp.float32))


def kernel(x, edge_index, W_fc0, b_fc0, W_conv, W_fc1, b_fc1):
    n, nfeat = x.shape
    nhid = W_fc0.shape[1]
    ncls = W_fc1.shape[1]
    e = edge_index.shape[1]
    npad = -(-n // (_NS * _CH)) * (_NS * _CH)
    epad = -(-e // (_NTILES * _CH)) * (_NTILES * _CH)

    src = edge_index[0]
    dst = edge_index[1]
    if epad != e:
        fill = jnp.full((epad - e,), npad - 1, dtype=jnp.int32)
        src = jnp.concatenate([src, fill])
        dst = jnp.concatenate([dst, fill])
    xp = jnp.zeros((npad, nfeat), x.dtype).at[:n].set(x)

    deg = _build_deg(npad, epad)(dst)
    degT = deg.T

    h, g = _build_fc0(npad, nfeat, nhid)(
        xp, W_fc0, b_fc0.reshape(1, nhid), degT)
    h0 = h
    spmm = _build_spmm(npad, epad, nhid)
    mid = _build_mid(npad, nhid)
    layer = _build_layer(npad, nhid)
    for _ in range(_NLAYERS):
        gi = g
        for _ in range(_HOP - 1):
            p = spmm(gi, src, dst)
            gi = mid(p, gi, degT)
        p = spmm(gi, src, dst)
        h, g = layer(p, gi, degT, h0, W_conv)

    ncpad = -(-ncls // 64) * 64
    Wf = jnp.zeros((nhid, ncpad), W_fc1.dtype).at[:, :ncls].set(W_fc1)
    bf = jnp.full((1, ncpad), -1e30, jnp.float32).at[0, :ncls].set(b_fc1)
    out = _build_final(npad, nhid, ncpad)(h, Wf, bf)
    return out[:n, :ncls]


# speedup vs baseline: 10.3167x; 1.0416x over previous
import functools

import jax
import jax.numpy as jnp
from jax import lax
from jax.experimental import pallas as pl
from jax.experimental.pallas import tpu as pltpu
from jax.experimental.pallas import tpu_sc as plsc

_ALPHA = 0.1
_HOP = 2
_NLAYERS = 4

_NC = 2
_NS = 16
_NTILES = _NC * _NS
_CH = 128


def _sc_mesh():
    return plsc.VectorSubcoreMesh(
        core_axis_name="c", subcore_axis_name="s",
        num_cores=_NC, num_subcores=_NS)


@functools.lru_cache(maxsize=None)
def _build_deg(npad, epad):
    ept = epad // _NTILES
    iters = ept // _CH
    rows_pt = npad // _NS

    def body(dst_hbm, out_hbm, didx_v, ones_v, zbuf_v, acc_sh):
        c = lax.axis_index("c")
        s = lax.axis_index("s")
        w = c * _NS + s
        one16 = jnp.full((16,), 1.0, jnp.float32)
        zero16 = jnp.zeros((16,), jnp.float32)
        for j in range(_CH // 16):
            ones_v[pl.ds(j * 16, 16)] = one16

        def zb(i, carry):
            zbuf_v[pl.ds(i * 16, 16)] = zero16
            return carry

        lax.fori_loop(0, rows_pt // 16, zb, 0)
        r0 = s * rows_pt
        pltpu.sync_copy(zbuf_v, acc_sh.at[pl.ds(r0, rows_pt)])
        pltpu.sync_copy(dst_hbm.at[pl.ds(w * iters, iters)], didx_v)
        plsc.subcore_barrier()

        def step(it, carry):
            pltpu.sync_copy(ones_v, acc_sh.at[didx_v.at[it]], add=True)
            return carry

        lax.fori_loop(0, iters, step, 0)
        plsc.subcore_barrier()
        pltpu.sync_copy(acc_sh.at[pl.ds(r0, rows_pt)],
                        out_hbm.at[c, pl.ds(r0, rows_pt)])

    return pl.kernel(
        body,
        out_type=jax.ShapeDtypeStruct((_NC, npad), jnp.float32),
        mesh=_sc_mesh(),
        compiler_params=pltpu.CompilerParams(use_tc_tiling_on_sc=False),
        scratch_types=[
            pltpu.VMEM((iters, _CH), jnp.int32),
            pltpu.VMEM((_CH,), jnp.float32),
            pltpu.VMEM((rows_pt,), jnp.float32),
            pltpu.VMEM_SHARED((npad,), jnp.float32),
        ],
    )


_NBUF = 4


@functools.lru_cache(maxsize=None)
def _build_spmm(npad, epad, nhid):
    ept = epad // _NTILES
    iters = ept // _CH
    groups = iters // _NBUF
    rows_pt = npad // _NS

    def body(g_hbm, src_hbm, dst_hbm, out_hbm, sidx_v, didx_v, rows_v,
             acc_sh, sems):
        c = lax.axis_index("c")
        s = lax.axis_index("s")
        w = c * _NS + s
        zero16 = jnp.zeros((16,), jnp.float32)

        def zb(i, carry):
            for j in range(nhid // 16):
                rows_v[0, i, pl.ds(j * 16, 16)] = zero16
            return carry

        lax.fori_loop(0, _CH, zb, 0)
        r0 = s * rows_pt
        for k in range(rows_pt // _CH):
            pltpu.sync_copy(rows_v.at[0], acc_sh.at[pl.ds(r0 + k * _CH, _CH)])
        row0 = w * iters
        pltpu.sync_copy(src_hbm.at[pl.ds(row0, iters)], sidx_v)
        pltpu.sync_copy(dst_hbm.at[pl.ds(row0, iters)], didx_v)
        plsc.subcore_barrier()

        def start_gather(it, b):
            pltpu.async_copy(g_hbm.at[sidx_v.at[it]], rows_v.at[b],
                             sems.at[b])

        def wait_gather(b):
            pltpu.make_async_copy(g_hbm.at[pl.ds(0, _CH)], rows_v.at[b],
                                  sems.at[b]).wait()

        for b in range(_NBUF - 1):
            start_gather(b, b)

        def grp(gi, carry):
            base_it = gi * _NBUF
            for b in range(_NBUF):
                it = base_it + b
                nxt = it + _NBUF - 1
                nb = (b + _NBUF - 1) % _NBUF

                @pl.when(nxt < iters)
                def _():
                    start_gather(nxt, nb)

                wait_gather(b)
                pltpu.sync_copy(rows_v.at[b], acc_sh.at[didx_v.at[it]],
                                add=True)
            return carry

        lax.fori_loop(0, groups, grp, 0)
        plsc.subcore_barrier()
        pltpu.sync_copy(acc_sh.at[pl.ds(r0, rows_pt)],
                        out_hbm.at[c, pl.ds(r0, rows_pt)])

    return pl.kernel(
        body,
        out_type=jax.ShapeDtypeStruct((_NC, npad, nhid), jnp.float32),
        mesh=_sc_mesh(),
        compiler_params=pltpu.CompilerParams(use_tc_tiling_on_sc=False),
        scratch_types=[
            pltpu.VMEM((iters, _CH), jnp.int32),
            pltpu.VMEM((iters, _CH), jnp.int32),
            pltpu.VMEM((_NBUF, _CH, nhid), jnp.float32),
            pltpu.VMEM_SHARED((npad, nhid), jnp.float32),
            pltpu.SemaphoreType.DMA((_NBUF,)),
        ],
    )


def _dinv_from(degT):
    deg = jnp.sum(degT, axis=1, keepdims=True) + 1.0
    return lax.rsqrt(jnp.maximum(deg, 1.0))


@functools.lru_cache(maxsize=None)
def _build_fc0(npad, nfeat, nhid):
    def body(x_ref, w_ref, b_ref, degT_ref, h_ref, g_ref):
        h = jnp.maximum(
            jnp.dot(x_ref[...], w_ref[...],
                    preferred_element_type=jnp.float32) + b_ref[...], 0.0)
        dinv = _dinv_from(degT_ref[...])
        h_ref[...] = h
        g_ref[...] = h * dinv

    return pl.pallas_call(
        body,
        out_shape=[jax.ShapeDtypeStruct((npad, nhid), jnp.float32)] * 2)


@functools.lru_cache(maxsize=None)
def _build_mid(npad, nhid):
    def body(p_ref, g_ref, degT_ref, o_ref):
        deg = jnp.sum(degT_ref[...], axis=1, keepdims=True) + 1.0
        inv = 1.0 / jnp.maximum(deg, 1.0)
        o_ref[...] = (p_ref[0] + p_ref[1] + g_ref[...]) * inv

    return pl.pallas_call(
        body,
        out_shape=jax.ShapeDtypeStruct((npad, nhid), jnp.float32))


@functools.lru_cache(maxsize=None)
def _build_layer(npad, nhid):
    def body(p_ref, g_ref, degT_ref, h0_ref, wc_ref, h_ref, gn_ref):
        dinv = _dinv_from(degT_ref[...])
        hi = (p_ref[0] + p_ref[1] + g_ref[...]) * dinv
        sup = (1.0 - _ALPHA) * hi + _ALPHA * h0_ref[...]
        h = jnp.maximum(
            jnp.dot(sup, wc_ref[...], preferred_element_type=jnp.float32),
            0.0)
        h_ref[...] = h
        gn_ref[...] = h * dinv

    return pl.pallas_call(
        body,
        out_shape=[jax.ShapeDtypeStruct((npad, nhid), jnp.float32)] * 2)


@functools.lru_cache(maxsize=None)
def _build_final(npad, nhid, ncpad):
    def body(h_ref, wf_ref, bf_ref, o_ref):
        logits = jnp.dot(h_ref[...], wf_ref[...],
                         preferred_element_type=jnp.float32) + bf_ref[...]
        m = jnp.max(logits, axis=1, keepdims=True)
        sh = logits - m
        lse = jnp.log(jnp.sum(jnp.exp(sh), axis=1, keepdims=True))
        o_ref[...] = sh - lse

    return pl.pallas_call(
        body,
        out_shape=jax.ShapeDtypeStruct((npad, ncpad), jnp.float32))


def kernel(x, edge_index, W_fc0, b_fc0, W_conv, W_fc1, b_fc1):
    n, nfeat = x.shape
    nhid = W_fc0.shape[1]
    ncls = W_fc1.shape[1]
    e = edge_index.shape[1]
    npad = -(-n // (_NS * _CH)) * (_NS * _CH)
    egrain = _NTILES * _CH * _NBUF
    epad = -(-e // egrain) * egrain

    src = edge_index[0]
    dst = edge_index[1]
    if epad != e:
        fill = jnp.full((epad - e,), npad - 1, dtype=jnp.int32)
        src = jnp.concatenate([src, fill])
        dst = jnp.concatenate([dst, fill])
    src = src.reshape(epad // _CH, _CH)
    dst = dst.reshape(epad // _CH, _CH)
    xp = jnp.zeros((npad, nfeat), x.dtype).at[:n].set(x)

    deg = _build_deg(npad, epad)(dst)
    degT = deg.T

    h, g = _build_fc0(npad, nfeat, nhid)(
        xp, W_fc0, b_fc0.reshape(1, nhid), degT)
    h0 = h
    spmm = _build_spmm(npad, epad, nhid)
    mid = _build_mid(npad, nhid)
    layer = _build_layer(npad, nhid)
    for _ in range(_NLAYERS):
        gi = g
        for _ in range(_HOP - 1):
            p = spmm(gi, src, dst)
            gi = mid(p, gi, degT)
        p = spmm(gi, src, dst)
        h, g = layer(p, gi, degT, h0, W_conv)

    ncpad = -(-ncls // 64) * 64
    Wf = jnp.zeros((nhid, ncpad), W_fc1.dtype).at[:, :ncls].set(W_fc1)
    bf = jnp.full((1, ncpad), -1e30, jnp.float32).at[0, :ncls].set(b_fc1)
    out = _build_final(npad, nhid, ncpad)(h, Wf, bf)
    return out[:n, :ncls]


# speedup vs baseline: 10.3560x; 1.0038x over previous
import functools

import jax
import jax.numpy as jnp
from jax import lax
from jax.experimental import pallas as pl
from jax.experimental.pallas import tpu as pltpu
from jax.experimental.pallas import tpu_sc as plsc

_ALPHA = 0.1
_HOP = 2
_NLAYERS = 4

_NC = 2
_NS = 16
_NTILES = _NC * _NS
_CH = 128


def _sc_mesh():
    return plsc.VectorSubcoreMesh(
        core_axis_name="c", subcore_axis_name="s",
        num_cores=_NC, num_subcores=_NS)


@functools.lru_cache(maxsize=None)
def _build_deg(npad, epad):
    ept = epad // _NTILES
    iters = ept // _CH
    rows_pt = npad // _NS

    def body(dst_hbm, out_hbm, didx_v, ones_v, zbuf_v, acc_sh):
        c = lax.axis_index("c")
        s = lax.axis_index("s")
        w = c * _NS + s
        one16 = jnp.full((16,), 1.0, jnp.float32)
        zero16 = jnp.zeros((16,), jnp.float32)
        for j in range(_CH // 16):
            ones_v[pl.ds(j * 16, 16)] = one16

        def zb(i, carry):
            zbuf_v[pl.ds(i * 16, 16)] = zero16
            return carry

        lax.fori_loop(0, rows_pt // 16, zb, 0)
        r0 = s * rows_pt
        pltpu.sync_copy(zbuf_v, acc_sh.at[pl.ds(r0, rows_pt)])
        pltpu.sync_copy(dst_hbm.at[pl.ds(w * iters, iters)], didx_v)
        plsc.subcore_barrier()

        def step(it, carry):
            pltpu.sync_copy(ones_v, acc_sh.at[didx_v.at[it]], add=True)
            return carry

        lax.fori_loop(0, iters, step, 0)
        plsc.subcore_barrier()
        pltpu.sync_copy(acc_sh.at[pl.ds(r0, rows_pt)],
                        out_hbm.at[c, pl.ds(r0, rows_pt)])

    return pl.kernel(
        body,
        out_type=jax.ShapeDtypeStruct((_NC, npad), jnp.float32),
        mesh=_sc_mesh(),
        compiler_params=pltpu.CompilerParams(use_tc_tiling_on_sc=False),
        scratch_types=[
            pltpu.VMEM((iters, _CH), jnp.int32),
            pltpu.VMEM((_CH,), jnp.float32),
            pltpu.VMEM((rows_pt,), jnp.float32),
            pltpu.VMEM_SHARED((npad,), jnp.float32),
        ],
    )


_NBUF = 8
_LEAD = 4


@functools.lru_cache(maxsize=None)
def _build_spmm(npad, epad, nhid):
    ept = epad // _NTILES
    iters = ept // _CH
    groups = iters // _NBUF
    rows_pt = npad // _NS

    def body(g_hbm, src_hbm, dst_hbm, out_hbm, sidx_v, didx_v, rows_v,
             acc_sh, gsems, ssems):
        c = lax.axis_index("c")
        s = lax.axis_index("s")
        w = c * _NS + s
        zero16 = jnp.zeros((16,), jnp.float32)

        def zb(i, carry):
            for j in range(nhid // 16):
                rows_v[0, i, pl.ds(j * 16, 16)] = zero16
            return carry

        lax.fori_loop(0, _CH, zb, 0)
        r0 = s * rows_pt
        for k in range(rows_pt // _CH):
            pltpu.sync_copy(rows_v.at[0], acc_sh.at[pl.ds(r0 + k * _CH, _CH)])
        row0 = w * iters
        pltpu.sync_copy(src_hbm.at[pl.ds(row0, iters)], sidx_v)
        pltpu.sync_copy(dst_hbm.at[pl.ds(row0, iters)], didx_v)
        plsc.subcore_barrier()

        def start_gather(it, b):
            pltpu.async_copy(g_hbm.at[sidx_v.at[it]], rows_v.at[b],
                             gsems.at[b])

        def wait_gather(b):
            pltpu.make_async_copy(g_hbm.at[pl.ds(0, _CH)], rows_v.at[b],
                                  gsems.at[b]).wait()

        def start_scatter(it, b):
            pltpu.async_copy(rows_v.at[b], acc_sh.at[didx_v.at[it]],
                             ssems.at[b], add=True)

        def wait_scatter(b):
            pltpu.make_async_copy(g_hbm.at[pl.ds(0, _CH)], rows_v.at[b],
                                  ssems.at[b]).wait()

        for k in range(_LEAD):
            start_gather(k, k % _NBUF)

        def grp(gi, carry):
            base_it = gi * _NBUF
            for b in range(_NBUF):
                it = base_it + b
                jt = it + _LEAD
                jb = (b + _LEAD) % _NBUF

                @pl.when(jt < iters)
                def _():
                    @pl.when(it >= _NBUF - _LEAD)
                    def _():
                        wait_scatter(jb)

                    start_gather(jt, jb)

                wait_gather(b)
                start_scatter(it, b)
            return carry

        lax.fori_loop(0, groups, grp, 0)
        for b in range(_NBUF):
            wait_scatter(b)
        plsc.subcore_barrier()
        pltpu.sync_copy(acc_sh.at[pl.ds(r0, rows_pt)],
                        out_hbm.at[c, pl.ds(r0, rows_pt)])

    return pl.kernel(
        body,
        out_type=jax.ShapeDtypeStruct((_NC, npad, nhid), jnp.float32),
        mesh=_sc_mesh(),
        compiler_params=pltpu.CompilerParams(use_tc_tiling_on_sc=False),
        scratch_types=[
            pltpu.VMEM((iters, _CH), jnp.int32),
            pltpu.VMEM((iters, _CH), jnp.int32),
            pltpu.VMEM((_NBUF, _CH, nhid), jnp.float32),
            pltpu.VMEM_SHARED((npad, nhid), jnp.float32),
            pltpu.SemaphoreType.DMA((_NBUF,)),
            pltpu.SemaphoreType.DMA((_NBUF,)),
        ],
    )


def _dinv_from(degT):
    deg = jnp.sum(degT, axis=1, keepdims=True) + 1.0
    return lax.rsqrt(jnp.maximum(deg, 1.0))


@functools.lru_cache(maxsize=None)
def _build_fc0(npad, nfeat, nhid):
    def body(x_ref, w_ref, b_ref, degT_ref, h_ref, g_ref):
        h = jnp.maximum(
            jnp.dot(x_ref[...], w_ref[...],
                    preferred_element_type=jnp.float32) + b_ref[...], 0.0)
        dinv = _dinv_from(degT_ref[...])
        h_ref[...] = h
        g_ref[...] = h * dinv

    return pl.pallas_call(
        body,
        out_shape=[jax.ShapeDtypeStruct((npad, nhid), jnp.float32)] * 2)


@functools.lru_cache(maxsize=None)
def _build_mid(npad, nhid):
    def body(p_ref, g_ref, degT_ref, o_ref):
        deg = jnp.sum(degT_ref[...], axis=1, keepdims=True) + 1.0
        inv = 1.0 / jnp.maximum(deg, 1.0)
        o_ref[...] = (p_ref[0] + p_ref[1] + g_ref[...]) * inv

    return pl.pallas_call(
        body,
        out_shape=jax.ShapeDtypeStruct((npad, nhid), jnp.float32))


@functools.lru_cache(maxsize=None)
def _build_layer(npad, nhid):
    def body(p_ref, g_ref, degT_ref, h0_ref, wc_ref, h_ref, gn_ref):
        dinv = _dinv_from(degT_ref[...])
        hi = (p_ref[0] + p_ref[1] + g_ref[...]) * dinv
        sup = (1.0 - _ALPHA) * hi + _ALPHA * h0_ref[...]
        h = jnp.maximum(
            jnp.dot(sup, wc_ref[...], preferred_element_type=jnp.float32),
            0.0)
        h_ref[...] = h
        gn_ref[...] = h * dinv

    return pl.pallas_call(
        body,
        out_shape=[jax.ShapeDtypeStruct((npad, nhid), jnp.float32)] * 2)


@functools.lru_cache(maxsize=None)
def _build_final(npad, nhid, ncpad):
    def body(h_ref, wf_ref, bf_ref, o_ref):
        logits = jnp.dot(h_ref[...], wf_ref[...],
                         preferred_element_type=jnp.float32) + bf_ref[...]
        m = jnp.max(logits, axis=1, keepdims=True)
        sh = logits - m
        lse = jnp.log(jnp.sum(jnp.exp(sh), axis=1, keepdims=True))
        o_ref[...] = sh - lse

    return pl.pallas_call(
        body,
        out_shape=jax.ShapeDtypeStruct((npad, ncpad), jnp.float32))


def kernel(x, edge_index, W_fc0, b_fc0, W_conv, W_fc1, b_fc1):
    n, nfeat = x.shape
    nhid = W_fc0.shape[1]
    ncls = W_fc1.shape[1]
    e = edge_index.shape[1]
    npad = -(-n // (_NS * _CH)) * (_NS * _CH)
    egrain = _NTILES * _CH * _NBUF
    epad = -(-e // egrain) * egrain

    src = edge_index[0]
    dst = edge_index[1]
    if epad != e:
        fill = jnp.full((epad - e,), npad - 1, dtype=jnp.int32)
        src = jnp.concatenate([src, fill])
        dst = jnp.concatenate([dst, fill])
    src = src.reshape(epad // _CH, _CH)
    dst = dst.reshape(epad // _CH, _CH)
    xp = jnp.zeros((npad, nfeat), x.dtype).at[:n].set(x)

    deg = _build_deg(npad, epad)(dst)
    degT = deg.T

    h, g = _build_fc0(npad, nfeat, nhid)(
        xp, W_fc0, b_fc0.reshape(1, nhid), degT)
    h0 = h
    spmm = _build_spmm(npad, epad, nhid)
    mid = _build_mid(npad, nhid)
    layer = _build_layer(npad, nhid)
    for _ in range(_NLAYERS):
        gi = g
        for _ in range(_HOP - 1):
            p = spmm(gi, src, dst)
            gi = mid(p, gi, degT)
        p = spmm(gi, src, dst)
        h, g = layer(p, gi, degT, h0, W_conv)

    ncpad = -(-ncls // 64) * 64
    Wf = jnp.zeros((nhid, ncpad), W_fc1.dtype).at[:, :ncls].set(W_fc1)
    bf = jnp.full((1, ncpad), -1e30, jnp.float32).at[0, :ncls].set(b_fc1)
    out = _build_final(npad, nhid, ncpad)(h, Wf, bf)
    return out[:n, :ncls]
